# baseline (speedup 1.0000x reference)
import functools

import jax
import jax.numpy as jnp
from jax import lax
from jax.experimental import pallas as pl
from jax.experimental.pallas import tpu as pltpu
from jax.experimental.pallas import tpu_sc as plsc

LANES = 16
NUM_WORKERS = 32
LN_EPS = 1e-12


def _lane_sum(x):
    idx = lax.iota(jnp.int32, LANES)
    dnums = lax.GatherDimensionNumbers(
        offset_dims=(), collapsed_slice_dims=(0,), start_index_map=(0,))
    for sh in (8, 4, 2, 1):
        perm = lax.gather(x, (idx ^ sh)[:, None], dimension_numbers=dnums,
                          slice_sizes=(1,),
                          mode=lax.GatherScatterMode.PROMISE_IN_BOUNDS)
        x = x + perm
    return x


def _rsqrt16(a):
    bits = lax.bitcast_convert_type(a, jnp.int32)
    seed = jnp.full((LANES,), 0x5F3759DF, jnp.int32) - (bits >> 1)
    y = lax.bitcast_convert_type(seed, jnp.float32)
    for _ in range(3):
        y = y * (1.5 - 0.5 * a * y * y)
    return y


def kernel(input_ids, token_table, pos_table, gamma, beta):
    B, S = input_ids.shape
    V, H = token_table.shape
    SW = S // NUM_WORKERS
    NH = H // LANES
    inv_h = 1.0 / H

    mesh = plsc.VectorSubcoreMesh(core_axis_name="c", subcore_axis_name="s")

    @functools.partial(
        pl.kernel,
        mesh=mesh,
        out_type=jax.ShapeDtypeStruct((B, S, H), jnp.float32),
        scratch_types=[
            pltpu.VMEM((B, SW), jnp.int32),
            pltpu.VMEM((SW, H), jnp.float32),
            pltpu.VMEM((H,), jnp.float32),
            pltpu.VMEM((H,), jnp.float32),
            pltpu.VMEM((SW, H), jnp.float32),
            pltpu.SemaphoreType.DMA,
        ],
    )
    def run(ids_h, tok_h, pos_h, g_h, bt_h, out_h,
            idx_v, pos_v, g_v, bt_v, rows_v, sem):
        wid = lax.axis_index("s") * 2 + lax.axis_index("c")
        s0 = wid * SW
        idx_descs = [
            pltpu.async_copy(ids_h.at[pl.ds(b * S + s0, SW)], idx_v.at[b], sem)
            for b in range(B)
        ]
        for d in idx_descs:
            d.wait()
        pltpu.sync_copy(pos_h.at[pl.ds(s0, SW)], pos_v)
        pltpu.sync_copy(g_h, g_v)
        pltpu.sync_copy(bt_h, bt_v)

        def body_b(b, carry):
            pltpu.async_copy(tok_h.at[idx_v.at[b]], rows_v, sem).wait()

            def body_r(r, inner):
                acc_s = jnp.zeros((LANES,), jnp.float32)
                acc_q = jnp.zeros((LANES,), jnp.float32)
                for i in range(NH):
                    sl = pl.ds(i * LANES, LANES)
                    x = rows_v[r, sl] + pos_v[r, sl]
                    rows_v[r, sl] = x
                    acc_s = acc_s + x
                    acc_q = acc_q + x * x
                mean = _lane_sum(acc_s) * inv_h
                msq = _lane_sum(acc_q) * inv_h
                var = jnp.maximum(msq - mean * mean, 0.0) + LN_EPS
                rstd = _rsqrt16(var)
                for i in range(NH):
                    sl = pl.ds(i * LANES, LANES)
                    x = rows_v[r, sl]
                    rows_v[r, sl] = (x - mean) * rstd * g_v[sl] + bt_v[sl]
                return inner

            lax.fori_loop(0, SW, body_r, 0)
            pltpu.sync_copy(rows_v, out_h.at[b, pl.ds(s0, SW)])
            return carry

        lax.fori_loop(0, B, body_b, 0)

    return run(input_ids.reshape(-1), token_table, pos_table, gamma, beta)


# speedup vs baseline: 1.2754x; 1.2754x over previous
import functools

import jax
import jax.numpy as jnp
from jax import lax
from jax.experimental import pallas as pl
from jax.experimental.pallas import tpu as pltpu
from jax.experimental.pallas import tpu_sc as plsc

LANES = 16
NUM_WORKERS = 32
LN_EPS = 1e-12


def _lane_sum(x):
    idx = lax.iota(jnp.int32, LANES)
    dnums = lax.GatherDimensionNumbers(
        offset_dims=(), collapsed_slice_dims=(0,), start_index_map=(0,))
    for sh in (8, 4, 2, 1):
        perm = lax.gather(x, (idx ^ sh)[:, None], dimension_numbers=dnums,
                          slice_sizes=(1,),
                          mode=lax.GatherScatterMode.PROMISE_IN_BOUNDS)
        x = x + perm
    return x


def _rsqrt16(a):
    bits = lax.bitcast_convert_type(a, jnp.int32)
    seed = jnp.full((LANES,), 0x5F3759DF, jnp.int32) - (bits >> 1)
    y = lax.bitcast_convert_type(seed, jnp.float32)
    for _ in range(3):
        y = y * (1.5 - 0.5 * a * y * y)
    return y


def kernel(input_ids, token_table, pos_table, gamma, beta):
    B, S = input_ids.shape
    V, H = token_table.shape
    SW = S // NUM_WORKERS
    NH = H // LANES
    inv_h = 1.0 / H

    mesh = plsc.VectorSubcoreMesh(core_axis_name="c", subcore_axis_name="s")

    @functools.partial(
        pl.kernel,
        mesh=mesh,
        out_type=jax.ShapeDtypeStruct((B, S, H), jnp.float32),
        scratch_types=[
            pltpu.VMEM((B, SW), jnp.int32),
            pltpu.VMEM((SW, H), jnp.float32),
            pltpu.VMEM((H,), jnp.float32),
            pltpu.VMEM((H,), jnp.float32),
            pltpu.VMEM((SW, H), jnp.float32),
            pltpu.VMEM((SW, H), jnp.float32),
            pltpu.VMEM((SW, H), jnp.float32),
            pltpu.VMEM((SW, H), jnp.float32),
            pltpu.SemaphoreType.DMA,
            pltpu.SemaphoreType.DMA,
            pltpu.SemaphoreType.DMA,
            pltpu.SemaphoreType.DMA,
            pltpu.SemaphoreType.DMA,
        ],
    )
    def run(ids_h, tok_h, pos_h, g_h, bt_h, out_h,
            idx_v, pos_v, g_v, bt_v, rows0, rows1, outb0, outb1,
            sem, semg0, semg1, semo0, semo1):
        wid = lax.axis_index("s") * 2 + lax.axis_index("c")
        s0 = wid * SW
        idx_descs = [
            pltpu.async_copy(ids_h.at[pl.ds(b * S + s0, SW)], idx_v.at[b], sem)
            for b in range(B)
        ]
        for d in idx_descs:
            d.wait()
        pltpu.sync_copy(pos_h.at[pl.ds(s0, SW)], pos_v)
        pltpu.sync_copy(g_h, g_v)
        pltpu.sync_copy(bt_h, bt_v)

        def gdesc(b, rows_ref, semg):
            return pltpu.make_async_copy(tok_h.at[idx_v.at[b]], rows_ref, semg)

        def odesc(b, outb_ref, semo):
            return pltpu.make_async_copy(outb_ref, out_h.at[b, pl.ds(s0, SW)],
                                         semo)

        def compute(rows_ref, out_ref):
            def body_r(r, inner):
                acc_s = jnp.zeros((LANES,), jnp.float32)
                acc_q = jnp.zeros((LANES,), jnp.float32)
                for i in range(NH):
                    sl = pl.ds(i * LANES, LANES)
                    x = rows_ref[r, sl] + pos_v[r, sl]
                    rows_ref[r, sl] = x
                    acc_s = acc_s + x
                    acc_q = acc_q + x * x
                mean = _lane_sum(acc_s) * inv_h
                msq = _lane_sum(acc_q) * inv_h
                var = jnp.maximum(msq - mean * mean, 0.0) + LN_EPS
                rstd = _rsqrt16(var)
                for i in range(NH):
                    sl = pl.ds(i * LANES, LANES)
                    x = rows_ref[r, sl]
                    out_ref[r, sl] = (x - mean) * rstd * g_v[sl] + bt_v[sl]
                return inner

            lax.fori_loop(0, SW, body_r, 0)

        gdesc(0, rows0, semg0).start()
        gdesc(1, rows1, semg1).start()
        n_groups = B // 2

        def group(g, carry):
            for b, rows_ref, outb_ref, semg, semo in (
                (2 * g, rows0, outb0, semg0, semo0),
                (2 * g + 1, rows1, outb1, semg1, semo1),
            ):
                gdesc(b, rows_ref, semg).wait()

                @pl.when(g > 0)
                def _drain():
                    odesc(b, outb_ref, semo).wait()

                compute(rows_ref, outb_ref)
                odesc(b, outb_ref, semo).start()

                @pl.when(g < n_groups - 1)
                def _prefetch():
                    gdesc(b + 2, rows_ref, semg).start()

            return carry

        lax.fori_loop(0, n_groups, group, 0)
        odesc(B - 2, outb0, semo0).wait()
        odesc(B - 1, outb1, semo1).wait()

    return run(input_ids.reshape(-1), token_table, pos_table, gamma, beta)
